# Initial kernel scaffold; baseline (speedup 1.0000x reference)
import jax
import jax.numpy as jnp
from jax.experimental import pallas as pl


def kernel(boxes, scores, feats, cls_W, cls_b):
    raise NotImplementedError("write your pallas kernel here")


# speedup vs baseline: 70.7330x; 70.7330x over previous
import jax
import jax.numpy as jnp
from jax.experimental import pallas as pl
from jax.experimental.pallas import tpu as pltpu

_N = 5000
_C = 128
_NCLS = 80
_K = 1000
_KP = 1024
_B = 256
_NB = 20
_NP = _NB * _B
_THR = 0.65
_F32 = jnp.float32
_HIGH = jax.lax.Precision.HIGHEST


def _iou(x1c, y1c, x2c, y2c, ac, x1r, y1r, x2r, y2r, ar):
    xx1 = jnp.maximum(x1c, x1r)
    yy1 = jnp.maximum(y1c, y1r)
    xx2 = jnp.minimum(x2c, x2r)
    yy2 = jnp.minimum(y2c, y2r)
    iw = jnp.maximum(xx2 - xx1, 0.0)
    ih = jnp.maximum(yy2 - yy1, 0.0)
    inter = iw * ih
    union = ac + ar - inter
    return inter / (union + 1e-9)


def _cols(b):
    x1, y1, x2, y2 = b[:, 0:1], b[:, 1:2], b[:, 2:3], b[:, 3:4]
    return x1, y1, x2, y2, (x2 - x1) * (y2 - y1)


def _rows(b):
    x1, y1, x2, y2 = b[0:1, :], b[1:2, :], b[2:3, :], b[3:4, :]
    return x1, y1, x2, y2, (x2 - x1) * (y2 - y1)


def _fixpoint(pre, M):
    def cond(c):
        return c[1]

    def body(c):
        kb, _ = c
        sup = jnp.dot(kb, M, preferred_element_type=_F32)
        new = pre * jnp.where(sup < 0.5, 1.0, 0.0).astype(_F32)
        return new, jnp.any(new != kb)

    kb, _ = jax.lax.while_loop(cond, body, (pre, jnp.bool_(True)))
    return kb


def _nms_kernel(bc3, br3, s3, feats, w, bias, out, keep, rank):
    lane = jax.lax.broadcasted_iota(jnp.int32, (1, _B), 1)
    rr = jax.lax.broadcasted_iota(jnp.int32, (_B, _B), 0)
    cc = jax.lax.broadcasted_iota(jnp.int32, (_B, _B), 1)
    upper = (rr < cc).astype(_F32)
    su = upper

    for i in range(_NB):
        keep[i] = jnp.ones((1, _B), _F32)

    for i in range(_NB):
        x1c, y1c, x2c, y2c, ac = _cols(br3[i])
        x1r, y1r, x2r, y2r, ar = _rows(bc3[i])
        iou_ii = _iou(x1c, y1c, x2c, y2c, ac, x1r, y1r, x2r, y2r, ar)
        M = jnp.where(iou_ii > _THR, 1.0, 0.0).astype(_F32) * upper
        valid = jnp.where(lane + i * _B < _N, 1.0, 0.0).astype(_F32)
        pre = keep[i] * valid
        kb = _fixpoint(pre, M)
        keep[i] = kb

        def sweep(j, carry, x1c=x1c, y1c=y1c, x2c=x2c, y2c=y2c, ac=ac, kb=kb):
            jx1, jy1, jx2, jy2, ja = _rows(bc3[j])
            iou_ij = _iou(x1c, y1c, x2c, y2c, ac, jx1, jy1, jx2, jy2, ja)
            g = jnp.where(iou_ij > _THR, 1.0, 0.0).astype(_F32)
            sup = jnp.dot(kb, g, preferred_element_type=_F32)
            keep[j] = keep[j] * jnp.where(sup < 0.5, 1.0, 0.0).astype(_F32)
            return carry

        if i + 1 < _NB:
            jax.lax.fori_loop(i + 1, _NB, sweep, 0)

    def nkbody(j, t):
        valid = jnp.where(lane + j * _B < _N, 1.0, 0.0).astype(_F32)
        return t + jnp.sum(keep[j] * valid)

    nk = jax.lax.fori_loop(0, _NB, nkbody, jnp.float32(0.0))

    def rbody(j, carry):
        runk, runs = carry
        valid = jnp.where(lane + j * _B < _N, 1.0, 0.0).astype(_F32)
        kp = keep[j] * valid
        sp = (1.0 - keep[j]) * valid
        prefk = jnp.dot(kp, su, preferred_element_type=_F32) + runk
        prefs = jnp.dot(sp, su, preferred_element_type=_F32) + runs
        rank[j] = jnp.where(kp > 0.5, prefk,
                            jnp.where(sp > 0.5, nk + prefs, 3.0e7))
        return runk + jnp.sum(kp), runs + jnp.sum(sp)

    jax.lax.fori_loop(0, _NB, rbody, (jnp.float32(0.0), jnp.float32(0.0)))

    for kc in range(_KP // _B):
        kcol = (jax.lax.broadcasted_iota(jnp.int32, (_B, 1), 0)
                .astype(_F32) + float(kc * _B))

        def obody(j, acc, kcol=kcol):
            conf = 1.0 / (1.0 + jnp.exp(-s3[j]))
            oh = jnp.where(rank[j] == kcol, 1.0, 0.0).astype(_F32) * conf
            fb = feats[pl.ds(j * _B, _B), :]
            return acc + jnp.dot(oh, fb, precision=_HIGH,
                                 preferred_element_type=_F32)

        acc = jax.lax.fori_loop(0, _NB, obody, jnp.zeros((_B, _C), _F32))
        res = jnp.dot(acc, w[:, :], precision=_HIGH,
                      preferred_element_type=_F32) + bias[0:1, :]
        out[kc * _B:(kc + 1) * _B, :] = res


def kernel(boxes, scores, feats, cls_W, cls_b):
    order = jnp.argsort(-scores)
    bs = boxes[order]
    ss = scores[order]
    fs = feats[order]
    pad = _NP - _N
    bsp = jnp.concatenate([bs, jnp.zeros((pad, 4), _F32)], axis=0)
    ssp = jnp.concatenate([ss, jnp.zeros((pad,), _F32)], axis=0)
    fsp = jnp.concatenate([fs, jnp.zeros((pad, _C), _F32)], axis=0)
    br3 = bsp.reshape(_NB, _B, 4)
    bc3 = jnp.transpose(br3, (0, 2, 1))
    s3 = ssp.reshape(_NB, 1, _B)
    out = pl.pallas_call(
        _nms_kernel,
        out_shape=jax.ShapeDtypeStruct((_KP, _NCLS), _F32),
        scratch_shapes=[
            pltpu.VMEM((_NB, 1, _B), _F32),
            pltpu.VMEM((_NB, 1, _B), _F32),
        ],
        interpret=False,
    )(bc3, br3, s3, fsp, cls_W, cls_b.reshape(1, _NCLS))
    return out[:_K]


# speedup vs baseline: 82.8308x; 1.1710x over previous
import jax
import jax.numpy as jnp
from jax.experimental import pallas as pl
from jax.experimental.pallas import tpu as pltpu

_N = 5000
_C = 128
_NCLS = 80
_K = 1000
_KP = 1024
_B = 256
_NB = 20
_NP = _NB * _B
_THR = 0.65
_F32 = jnp.float32
_HIGH = jax.lax.Precision.HIGHEST


_S = _THR / (1.0 + _THR)


def _sup_mask(x1c, y1c, x2c, y2c, uc, x1r, y1r, x2r, y2r, ur):
    xx1 = jnp.maximum(x1c, x1r)
    yy1 = jnp.maximum(y1c, y1r)
    xx2 = jnp.minimum(x2c, x2r)
    yy2 = jnp.minimum(y2c, y2r)
    iw = jnp.maximum(xx2 - xx1, 0.0)
    ih = jnp.maximum(yy2 - yy1, 0.0)
    inter = iw * ih
    return jnp.where(inter > uc + ur, 1.0, 0.0).astype(_F32)


def _cols(b):
    x1, y1, x2, y2 = b[:, 0:1], b[:, 1:2], b[:, 2:3], b[:, 3:4]
    return x1, y1, x2, y2, _S * ((x2 - x1) * (y2 - y1) + 0.5e-9)


def _rows(b):
    x1, y1, x2, y2 = b[0:1, :], b[1:2, :], b[2:3, :], b[3:4, :]
    return x1, y1, x2, y2, _S * ((x2 - x1) * (y2 - y1) + 0.5e-9)


def _fixpoint(pre, M):
    def cond(c):
        return c[1]

    def body(c):
        kb, _ = c
        sup = jnp.dot(kb, M, preferred_element_type=_F32)
        new = pre * jnp.where(sup < 0.5, 1.0, 0.0).astype(_F32)
        return new, jnp.any(new != kb)

    kb, _ = jax.lax.while_loop(cond, body, (pre, jnp.bool_(True)))
    return kb


def _nms_kernel(bc3, br3, s3, feats, w, bias, out, keep, rank):
    lane = jax.lax.broadcasted_iota(jnp.int32, (1, _B), 1)
    rr = jax.lax.broadcasted_iota(jnp.int32, (_B, _B), 0)
    cc = jax.lax.broadcasted_iota(jnp.int32, (_B, _B), 1)
    upper = (rr < cc).astype(_F32)
    su = upper

    for i in range(_NB):
        keep[i] = jnp.ones((1, _B), _F32)

    for i in range(_NB):
        x1c, y1c, x2c, y2c, ac = _cols(br3[i])
        x1r, y1r, x2r, y2r, ar = _rows(bc3[i])
        M = _sup_mask(x1c, y1c, x2c, y2c, ac,
                      x1r, y1r, x2r, y2r, ar) * upper
        valid = jnp.where(lane + i * _B < _N, 1.0, 0.0).astype(_F32)
        pre = keep[i] * valid
        kb = _fixpoint(pre, M)
        keep[i] = kb

        def sweep(j, carry, x1c=x1c, y1c=y1c, x2c=x2c, y2c=y2c, ac=ac, kb=kb):
            jx1, jy1, jx2, jy2, ja = _rows(bc3[j])
            g = _sup_mask(x1c, y1c, x2c, y2c, ac, jx1, jy1, jx2, jy2, ja)
            sup = jnp.dot(kb, g, preferred_element_type=_F32)
            keep[j] = keep[j] * jnp.where(sup < 0.5, 1.0, 0.0).astype(_F32)
            return carry

        if i + 1 < _NB:
            jax.lax.fori_loop(i + 1, _NB, sweep, 0)

    def nkbody(j, t):
        valid = jnp.where(lane + j * _B < _N, 1.0, 0.0).astype(_F32)
        return t + jnp.sum(keep[j] * valid)

    nk = jax.lax.fori_loop(0, _NB, nkbody, jnp.float32(0.0))

    def rbody(j, carry):
        runk, runs = carry
        valid = jnp.where(lane + j * _B < _N, 1.0, 0.0).astype(_F32)
        kp = keep[j] * valid
        sp = (1.0 - keep[j]) * valid
        prefk = jnp.dot(kp, su, preferred_element_type=_F32) + runk
        prefs = jnp.dot(sp, su, preferred_element_type=_F32) + runs
        rank[j] = jnp.where(kp > 0.5, prefk,
                            jnp.where(sp > 0.5, nk + prefs, 3.0e7))
        return runk + jnp.sum(kp), runs + jnp.sum(sp)

    jax.lax.fori_loop(0, _NB, rbody, (jnp.float32(0.0), jnp.float32(0.0)))

    for kc in range(_KP // _B):
        kcol = (jax.lax.broadcasted_iota(jnp.int32, (_B, 1), 0)
                .astype(_F32) + float(kc * _B))

        def obody(j, acc, kcol=kcol):
            conf = 1.0 / (1.0 + jnp.exp(-s3[j]))
            oh = jnp.where(rank[j] == kcol, 1.0, 0.0).astype(_F32) * conf
            fb = feats[pl.ds(j * _B, _B), :]
            return acc + jnp.dot(oh, fb, preferred_element_type=_F32)

        acc = jax.lax.fori_loop(0, _NB, obody, jnp.zeros((_B, _C), _F32))
        res = jnp.dot(acc, w[:, :],
                      preferred_element_type=_F32) + bias[0:1, :]
        out[kc * _B:(kc + 1) * _B, :] = res


def kernel(boxes, scores, feats, cls_W, cls_b):
    order = jnp.argsort(-scores)
    bs = boxes[order]
    ss = scores[order]
    fs = feats[order]
    pad = _NP - _N
    bsp = jnp.concatenate([bs, jnp.zeros((pad, 4), _F32)], axis=0)
    ssp = jnp.concatenate([ss, jnp.zeros((pad,), _F32)], axis=0)
    fsp = jnp.concatenate([fs, jnp.zeros((pad, _C), _F32)], axis=0)
    br3 = bsp.reshape(_NB, _B, 4)
    bc3 = jnp.transpose(br3, (0, 2, 1))
    s3 = ssp.reshape(_NB, 1, _B)
    out = pl.pallas_call(
        _nms_kernel,
        out_shape=jax.ShapeDtypeStruct((_KP, _NCLS), _F32),
        scratch_shapes=[
            pltpu.VMEM((_NB, 1, _B), _F32),
            pltpu.VMEM((_NB, 1, _B), _F32),
        ],
        interpret=False,
    )(bc3, br3, s3, fsp, cls_W, cls_b.reshape(1, _NCLS))
    return out[:_K]


# speedup vs baseline: 108.8101x; 1.3136x over previous
import jax
import jax.numpy as jnp
from jax.experimental import pallas as pl
from jax.experimental.pallas import tpu as pltpu

_N = 5000
_C = 128
_NCLS = 80
_K = 1000
_KP = 1024
_B = 256
_NB = 20
_NP = _NB * _B
_THR = 0.65
_F32 = jnp.float32
_HIGH = jax.lax.Precision.HIGHEST


_S = _THR / (1.0 + _THR)


def _sup_mask(x1c, y1c, x2c, y2c, uc, x1r, y1r, x2r, y2r, ur):
    xx1 = jnp.maximum(x1c, x1r)
    yy1 = jnp.maximum(y1c, y1r)
    xx2 = jnp.minimum(x2c, x2r)
    yy2 = jnp.minimum(y2c, y2r)
    iw = jnp.maximum(xx2 - xx1, 0.0)
    ih = jnp.maximum(yy2 - yy1, 0.0)
    inter = iw * ih
    return jnp.where(inter > uc + ur, 1.0, 0.0).astype(_F32)


def _cols(b):
    x1, y1, x2, y2 = b[:, 0:1], b[:, 1:2], b[:, 2:3], b[:, 3:4]
    return x1, y1, x2, y2, _S * ((x2 - x1) * (y2 - y1) + 0.5e-9)


def _rows(b):
    x1, y1, x2, y2 = b[0:1, :], b[1:2, :], b[2:3, :], b[3:4, :]
    return x1, y1, x2, y2, _S * ((x2 - x1) * (y2 - y1) + 0.5e-9)


def _fixpoint(pre, M):
    def cond(c):
        return c[1]

    def body(c):
        kb, _ = c
        sup = jnp.dot(kb, M, preferred_element_type=_F32)
        new = pre * jnp.where(sup < 0.5, 1.0, 0.0).astype(_F32)
        return new, jnp.any(new != kb)

    kb, _ = jax.lax.while_loop(cond, body, (pre, jnp.bool_(True)))
    return kb


def _nms_kernel(bc3, br3, s3, feats, w, bias, out, keep, rank2, conf2):
    lane = jax.lax.broadcasted_iota(jnp.int32, (1, _B), 1)
    rr = jax.lax.broadcasted_iota(jnp.int32, (_B, _B), 0)
    cc = jax.lax.broadcasted_iota(jnp.int32, (_B, _B), 1)
    upper = (rr < cc).astype(_F32)
    su = upper

    for i in range(_NB):
        keep[i] = jnp.ones((1, _B), _F32)

    for i in range(_NB):
        x1c, y1c, x2c, y2c, ac = _cols(br3[i])
        x1r, y1r, x2r, y2r, ar = _rows(bc3[i])
        M = _sup_mask(x1c, y1c, x2c, y2c, ac,
                      x1r, y1r, x2r, y2r, ar) * upper
        valid = jnp.where(lane + i * _B < _N, 1.0, 0.0).astype(_F32)
        pre = keep[i] * valid
        kb = _fixpoint(pre, M)
        keep[i] = kb

        def sweep1(j, x1c=x1c, y1c=y1c, x2c=x2c, y2c=y2c, ac=ac, kb=kb):
            jx1, jy1, jx2, jy2, ja = _rows(bc3[j])
            g = _sup_mask(x1c, y1c, x2c, y2c, ac, jx1, jy1, jx2, jy2, ja)
            sup = jnp.dot(kb, g, preferred_element_type=_F32)
            keep[j] = keep[j] * jnp.where(sup < 0.5, 1.0, 0.0).astype(_F32)

        nj = _NB - 1 - i
        if nj > 0:
            def sweep2(t, carry, i=i, sweep1=sweep1):
                j = i + 1 + 2 * t
                sweep1(j)
                sweep1(j + 1)
                return carry

            if nj // 2 > 0:
                jax.lax.fori_loop(0, nj // 2, sweep2, 0)
            if nj % 2 == 1:
                sweep1(_NB - 1)

    def nkbody(j, t):
        valid = jnp.where(lane + j * _B < _N, 1.0, 0.0).astype(_F32)
        return t + jnp.sum(keep[j] * valid)

    nk = jax.lax.fori_loop(0, _NB, nkbody, jnp.float32(0.0))

    runk = jnp.float32(0.0)
    runs = jnp.float32(0.0)
    for j in range(_NB):
        valid = jnp.where(lane + j * _B < _N, 1.0, 0.0).astype(_F32)
        kp = keep[j] * valid
        sp = (1.0 - keep[j]) * valid
        prefk = jnp.dot(kp, su, preferred_element_type=_F32) + runk
        prefs = jnp.dot(sp, su, preferred_element_type=_F32) + runs
        rank2[0:1, j * _B:(j + 1) * _B] = jnp.where(
            kp > 0.5, prefk, jnp.where(sp > 0.5, nk + prefs, 3.0e7))
        conf2[0:1, j * _B:(j + 1) * _B] = 1.0 / (1.0 + jnp.exp(-s3[j]))
        runk = runk + jnp.sum(kp)
        runs = runs + jnp.sum(sp)

    rrow = rank2[0:1, :]
    crow = conf2[0:1, :]
    fall = feats[:, :]
    for kc in range(_KP // _B):
        kcol = (jax.lax.broadcasted_iota(jnp.int32, (_B, 1), 0)
                .astype(_F32) + float(kc * _B))
        oh = jnp.where(rrow == kcol, 1.0, 0.0).astype(_F32) * crow
        acc = jnp.dot(oh, fall, preferred_element_type=_F32)
        res = jnp.dot(acc, w[:, :],
                      preferred_element_type=_F32) + bias[0:1, :]
        out[kc * _B:(kc + 1) * _B, :] = res


def kernel(boxes, scores, feats, cls_W, cls_b):
    order = jnp.argsort(-scores)
    packed = jnp.concatenate([feats, boxes, scores[:, None]], axis=1)
    ps = packed[order]
    fs = ps[:, :_C]
    bs = ps[:, _C:_C + 4]
    ss = ps[:, _C + 4]
    pad = _NP - _N
    bsp = jnp.concatenate([bs, jnp.zeros((pad, 4), _F32)], axis=0)
    ssp = jnp.concatenate([ss, jnp.zeros((pad,), _F32)], axis=0)
    fsp = jnp.concatenate([fs, jnp.zeros((pad, _C), _F32)], axis=0)
    br3 = bsp.reshape(_NB, _B, 4)
    bc3 = jnp.transpose(br3, (0, 2, 1))
    s3 = ssp.reshape(_NB, 1, _B)
    out = pl.pallas_call(
        _nms_kernel,
        out_shape=jax.ShapeDtypeStruct((_KP, _NCLS), _F32),
        scratch_shapes=[
            pltpu.VMEM((_NB, 1, _B), _F32),
            pltpu.VMEM((1, _NP), _F32),
            pltpu.VMEM((1, _NP), _F32),
        ],
        interpret=False,
    )(bc3, br3, s3, fsp, cls_W, cls_b.reshape(1, _NCLS))
    return out[:_K]


# speedup vs baseline: 112.3665x; 1.0327x over previous
import jax
import jax.numpy as jnp
from jax.experimental import pallas as pl
from jax.experimental.pallas import tpu as pltpu

_N = 5000
_C = 128
_NCLS = 80
_K = 1000
_KP = 1024
_B = 256
_NB = 20
_NP = _NB * _B
_THR = 0.65
_F32 = jnp.float32
_HIGH = jax.lax.Precision.HIGHEST


_S = _THR / (1.0 + _THR)


def _sup_mask(x1c, y1c, x2c, y2c, uc, x1r, y1r, x2r, y2r, ur):
    xx1 = jnp.maximum(x1c, x1r)
    yy1 = jnp.maximum(y1c, y1r)
    xx2 = jnp.minimum(x2c, x2r)
    yy2 = jnp.minimum(y2c, y2r)
    iw = jnp.maximum(xx2 - xx1, 0.0)
    ih = yy2 - yy1
    inter = iw * ih
    return (inter > uc + ur).astype(jnp.bfloat16)


def _cols(b):
    x1, y1, x2, y2 = b[:, 0:1], b[:, 1:2], b[:, 2:3], b[:, 3:4]
    return x1, y1, x2, y2, _S * ((x2 - x1) * (y2 - y1) + 0.5e-9)


def _rows(b):
    x1, y1, x2, y2 = b[0:1, :], b[1:2, :], b[2:3, :], b[3:4, :]
    return x1, y1, x2, y2, _S * ((x2 - x1) * (y2 - y1) + 0.5e-9)


def _fixpoint(pre, M):
    def cond(c):
        return c[1]

    def body(c):
        kb, _ = c
        sup = jnp.dot(kb, M, preferred_element_type=_F32)
        new = pre * (sup < 0.5).astype(jnp.bfloat16)
        diff = (new - kb).astype(_F32)
        return new, jnp.sum(diff * diff) > 0.5

    kb, _ = jax.lax.while_loop(cond, body, (pre, jnp.bool_(True)))
    return kb


def _nms_kernel(bc3, br3, s3, feats, w, bias, out, keep, rank2, conf2):
    lane = jax.lax.broadcasted_iota(jnp.int32, (1, _B), 1)
    rr = jax.lax.broadcasted_iota(jnp.int32, (_B, _B), 0)
    cc = jax.lax.broadcasted_iota(jnp.int32, (_B, _B), 1)
    upper = (rr < cc).astype(jnp.bfloat16)
    su = upper

    for i in range(_NB):
        keep[i] = jnp.ones((1, _B), _F32)

    for i in range(_NB):
        x1c, y1c, x2c, y2c, ac = _cols(br3[i])
        x1r, y1r, x2r, y2r, ar = _rows(bc3[i])
        M = _sup_mask(x1c, y1c, x2c, y2c, ac,
                      x1r, y1r, x2r, y2r, ar) * upper
        valid = (lane + i * _B < _N).astype(jnp.bfloat16)
        pre = keep[i].astype(jnp.bfloat16) * valid
        kb = _fixpoint(pre, M)
        keep[i] = kb.astype(_F32)

        def sweep1(j, x1c=x1c, y1c=y1c, x2c=x2c, y2c=y2c, ac=ac, kb=kb):
            jx1, jy1, jx2, jy2, ja = _rows(bc3[j])
            g = _sup_mask(x1c, y1c, x2c, y2c, ac, jx1, jy1, jx2, jy2, ja)
            sup = jnp.dot(kb, g, preferred_element_type=_F32)
            keep[j] = keep[j] * jnp.where(sup < 0.5, 1.0, 0.0).astype(_F32)

        nj = _NB - 1 - i
        if nj > 0:
            def sweep2(t, carry, i=i, sweep1=sweep1):
                j = i + 1 + 2 * t
                sweep1(j)
                sweep1(j + 1)
                return carry

            if nj // 2 > 0:
                jax.lax.fori_loop(0, nj // 2, sweep2, 0)
            if nj % 2 == 1:
                sweep1(_NB - 1)

    def nkbody(j, t):
        valid = jnp.where(lane + j * _B < _N, 1.0, 0.0).astype(_F32)
        return t + jnp.sum(keep[j] * valid)

    nk = jax.lax.fori_loop(0, _NB, nkbody, jnp.float32(0.0))

    runk = jnp.float32(0.0)
    runs = jnp.float32(0.0)
    for j in range(_NB):
        valid = jnp.where(lane + j * _B < _N, 1.0, 0.0).astype(_F32)
        kp = keep[j] * valid
        sp = (1.0 - keep[j]) * valid
        prefk = jnp.dot(kp.astype(jnp.bfloat16), su,
                        preferred_element_type=_F32) + runk
        prefs = jnp.dot(sp.astype(jnp.bfloat16), su,
                        preferred_element_type=_F32) + runs
        rank2[0:1, j * _B:(j + 1) * _B] = jnp.where(
            kp > 0.5, prefk, jnp.where(sp > 0.5, nk + prefs, 3.0e7))
        conf2[0:1, j * _B:(j + 1) * _B] = 1.0 / (1.0 + jnp.exp(-s3[j]))
        runk = runk + jnp.sum(kp)
        runs = runs + jnp.sum(sp)

    rrow = rank2[0:1, :]
    crow = conf2[0:1, :]
    fall = feats[:, :].astype(jnp.bfloat16)
    for kc in range(_KP // _B):
        kcol = (jax.lax.broadcasted_iota(jnp.int32, (_B, 1), 0)
                .astype(_F32) + float(kc * _B))
        oh = ((rrow == kcol).astype(_F32) * crow).astype(jnp.bfloat16)
        acc = jnp.dot(oh, fall, preferred_element_type=_F32)
        res = jnp.dot(acc, w[:, :],
                      preferred_element_type=_F32) + bias[0:1, :]
        out[kc * _B:(kc + 1) * _B, :] = res


def kernel(boxes, scores, feats, cls_W, cls_b):
    order = jnp.argsort(-scores)
    packed = jnp.concatenate([feats, boxes, scores[:, None]], axis=1)
    ps = packed[order]
    fs = ps[:, :_C]
    bs = ps[:, _C:_C + 4]
    ss = ps[:, _C + 4]
    pad = _NP - _N
    bsp = jnp.concatenate([bs, jnp.zeros((pad, 4), _F32)], axis=0)
    ssp = jnp.concatenate([ss, jnp.zeros((pad,), _F32)], axis=0)
    fsp = jnp.concatenate([fs, jnp.zeros((pad, _C), _F32)], axis=0)
    br3 = bsp.reshape(_NB, _B, 4)
    bc3 = jnp.transpose(br3, (0, 2, 1))
    s3 = ssp.reshape(_NB, 1, _B)
    out = pl.pallas_call(
        _nms_kernel,
        out_shape=jax.ShapeDtypeStruct((_KP, _NCLS), _F32),
        scratch_shapes=[
            pltpu.VMEM((_NB, 1, _B), _F32),
            pltpu.VMEM((1, _NP), _F32),
            pltpu.VMEM((1, _NP), _F32),
        ],
        interpret=False,
    )(bc3, br3, s3, fsp, cls_W, cls_b.reshape(1, _NCLS))
    return out[:_K]


# speedup vs baseline: 115.2285x; 1.0255x over previous
import jax
import jax.numpy as jnp
from jax.experimental import pallas as pl
from jax.experimental.pallas import tpu as pltpu

_N = 5000
_C = 128
_NCLS = 80
_K = 1000
_KP = 1024
_B = 256
_NB = 20
_NP = _NB * _B
_THR = 0.65
_F32 = jnp.float32
_HIGH = jax.lax.Precision.HIGHEST


_S = _THR / (1.0 + _THR)


def _sup_mask(x1c, y1c, x2c, y2c, uc, x1r, y1r, x2r, y2r, ur):
    xx1 = jnp.maximum(x1c, x1r)
    yy1 = jnp.maximum(y1c, y1r)
    xx2 = jnp.minimum(x2c, x2r)
    yy2 = jnp.minimum(y2c, y2r)
    iw = jnp.maximum(xx2 - xx1, 0.0)
    ih = yy2 - yy1
    inter = iw * ih
    return (inter > uc + ur).astype(jnp.bfloat16)


def _cols(b):
    x1, y1, x2, y2 = b[:, 0:1], b[:, 1:2], b[:, 2:3], b[:, 3:4]
    return x1, y1, x2, y2, _S * ((x2 - x1) * (y2 - y1) + 0.5e-9)


def _rows(b):
    x1, y1, x2, y2 = b[0:1, :], b[1:2, :], b[2:3, :], b[3:4, :]
    return x1, y1, x2, y2, _S * ((x2 - x1) * (y2 - y1) + 0.5e-9)


def _fixpoint(pre, M):
    def cond(c):
        return c[1]

    def body(c):
        kb, _ = c
        sup = jnp.dot(kb, M, preferred_element_type=_F32)
        new = pre * (sup < 0.5).astype(jnp.bfloat16)
        diff = (new - kb).astype(_F32)
        return new, jnp.sum(diff * diff) > 0.5

    kb, _ = jax.lax.while_loop(cond, body, (pre, jnp.bool_(True)))
    return kb


def _nms_kernel(bc3, br3, s3, feats, w, bias, out, keep, rank2, conf2):
    lane = jax.lax.broadcasted_iota(jnp.int32, (1, _B), 1)
    rr = jax.lax.broadcasted_iota(jnp.int32, (_B, _B), 0)
    cc = jax.lax.broadcasted_iota(jnp.int32, (_B, _B), 1)
    upper = (rr < cc).astype(jnp.bfloat16)
    su = upper

    for i in range(_NB):
        keep[i] = jnp.ones((1, _B), _F32)

    for i in range(_NB):
        x1c, y1c, x2c, y2c, ac = _cols(br3[i])
        x1r, y1r, x2r, y2r, ar = _rows(bc3[i])
        M = _sup_mask(x1c, y1c, x2c, y2c, ac,
                      x1r, y1r, x2r, y2r, ar) * upper
        valid = (lane + i * _B < _N).astype(jnp.bfloat16)
        pre = keep[i].astype(jnp.bfloat16) * valid
        kb = _fixpoint(pre, M)
        keep[i] = kb.astype(_F32)

        def sweep1(j, x1c=x1c, y1c=y1c, x2c=x2c, y2c=y2c, ac=ac, kb=kb):
            jx1, jy1, jx2, jy2, ja = _rows(bc3[j])
            g = _sup_mask(x1c, y1c, x2c, y2c, ac, jx1, jy1, jx2, jy2, ja)
            sup = jnp.dot(kb, g, preferred_element_type=_F32)
            keep[j] = keep[j] * jnp.where(sup < 0.5, 1.0, 0.0).astype(_F32)

        nj = _NB - 1 - i
        un = 4
        if nj > 0:
            def sweepu(t, carry, i=i, sweep1=sweep1):
                base = i + 1 + un * t
                for u in range(un):
                    sweep1(base + u)
                return carry

            if nj // un > 0:
                jax.lax.fori_loop(0, nj // un, sweepu, 0)
            for u in range(nj % un):
                sweep1(i + 1 + un * (nj // un) + u)

    def nkbody(j, t):
        valid = jnp.where(lane + j * _B < _N, 1.0, 0.0).astype(_F32)
        return t + jnp.sum(keep[j] * valid)

    nk = jax.lax.fori_loop(0, _NB, nkbody, jnp.float32(0.0))

    runk = jnp.float32(0.0)
    runs = jnp.float32(0.0)
    for j in range(_NB):
        valid = jnp.where(lane + j * _B < _N, 1.0, 0.0).astype(_F32)
        kp = keep[j] * valid
        sp = (1.0 - keep[j]) * valid
        prefk = jnp.dot(kp.astype(jnp.bfloat16), su,
                        preferred_element_type=_F32) + runk
        prefs = jnp.dot(sp.astype(jnp.bfloat16), su,
                        preferred_element_type=_F32) + runs
        rank2[0:1, j * _B:(j + 1) * _B] = jnp.where(
            kp > 0.5, prefk, jnp.where(sp > 0.5, nk + prefs, 3.0e7))
        conf2[0:1, j * _B:(j + 1) * _B] = 1.0 / (1.0 + jnp.exp(-s3[j]))
        runk = runk + jnp.sum(kp)
        runs = runs + jnp.sum(sp)

    rrow = rank2[0:1, :]
    crow = conf2[0:1, :]
    fall = feats[:, :]
    for kc in range(_KP // _B):
        kcol = (jax.lax.broadcasted_iota(jnp.int32, (_B, 1), 0)
                .astype(_F32) + float(kc * _B))
        oh = ((rrow == kcol).astype(_F32) * crow).astype(jnp.bfloat16)
        acc = jnp.dot(oh, fall, preferred_element_type=_F32)
        res = jnp.dot(acc, w[:, :],
                      preferred_element_type=_F32) + bias[0:1, :]
        out[kc * _B:(kc + 1) * _B, :] = res


def kernel(boxes, scores, feats, cls_W, cls_b):
    order = jnp.argsort(-scores)
    f16 = feats.astype(jnp.bfloat16)
    fpk = jax.lax.bitcast_convert_type(f16.reshape(_N, _C // 2, 2), _F32)
    packed = jnp.concatenate([fpk, boxes, scores[:, None]], axis=1)
    ps = packed[order]
    fs = jax.lax.bitcast_convert_type(ps[:, :_C // 2],
                                      jnp.bfloat16).reshape(_N, _C)
    bs = ps[:, _C // 2:_C // 2 + 4]
    ss = ps[:, _C // 2 + 4]
    pad = _NP - _N
    bsp = jnp.concatenate([bs, jnp.zeros((pad, 4), _F32)], axis=0)
    ssp = jnp.concatenate([ss, jnp.zeros((pad,), _F32)], axis=0)
    fsp = jnp.concatenate([fs, jnp.zeros((pad, _C), jnp.bfloat16)], axis=0)
    br3 = bsp.reshape(_NB, _B, 4)
    bc3 = jnp.transpose(br3, (0, 2, 1))
    s3 = ssp.reshape(_NB, 1, _B)
    out = pl.pallas_call(
        _nms_kernel,
        out_shape=jax.ShapeDtypeStruct((_KP, _NCLS), _F32),
        scratch_shapes=[
            pltpu.VMEM((_NB, 1, _B), _F32),
            pltpu.VMEM((1, _NP), _F32),
            pltpu.VMEM((1, _NP), _F32),
        ],
        interpret=False,
    )(bc3, br3, s3, fsp, cls_W, cls_b.reshape(1, _NCLS))
    return out[:_K]


# speedup vs baseline: 118.4992x; 1.0284x over previous
import jax
import jax.numpy as jnp
from jax.experimental import pallas as pl
from jax.experimental.pallas import tpu as pltpu

_N = 5000
_C = 128
_NCLS = 80
_K = 1000
_KP = 1024
_B = 256
_NB = 20
_NP = _NB * _B
_THR = 0.65
_F32 = jnp.float32
_HIGH = jax.lax.Precision.HIGHEST


_S = _THR / (1.0 + _THR)


def _sup_mask(x1c, y1c, x2c, y2c, uc, x1r, y1r, x2r, y2r, ur):
    xx1 = jnp.maximum(x1c, x1r)
    yy1 = jnp.maximum(y1c, y1r)
    xx2 = jnp.minimum(x2c, x2r)
    yy2 = jnp.minimum(y2c, y2r)
    iw = jnp.maximum(xx2 - xx1, 0.0)
    ih = yy2 - yy1
    inter = iw * ih
    return (inter > uc + ur).astype(jnp.bfloat16)


def _cols(b):
    x1, y1, x2, y2 = b[:, 0:1], b[:, 1:2], b[:, 2:3], b[:, 3:4]
    return x1, y1, x2, y2, _S * ((x2 - x1) * (y2 - y1) + 0.5e-9)


def _rows(b):
    x1, y1, x2, y2 = b[0:1, :], b[1:2, :], b[2:3, :], b[3:4, :]
    return x1, y1, x2, y2, _S * ((x2 - x1) * (y2 - y1) + 0.5e-9)


def _fixpoint(pre, M):
    def cond(c):
        return c[1]

    def body(c):
        kb, _ = c
        sup = jnp.dot(kb, M, preferred_element_type=_F32)
        new = pre * (sup < 0.5).astype(jnp.bfloat16)
        diff = (new - kb).astype(_F32)
        return new, jnp.sum(diff * diff) > 0.5

    kb, _ = jax.lax.while_loop(cond, body, (pre, jnp.bool_(True)))
    return kb


def _nms_kernel(bc3, br3, s3, rank2, conf2, keep):
    lane = jax.lax.broadcasted_iota(jnp.int32, (1, _B), 1)
    rr = jax.lax.broadcasted_iota(jnp.int32, (_B, _B), 0)
    cc = jax.lax.broadcasted_iota(jnp.int32, (_B, _B), 1)
    upper = (rr < cc).astype(jnp.bfloat16)
    su = upper

    for i in range(_NB):
        keep[i] = jnp.ones((1, _B), _F32)

    for i in range(_NB):
        x1c, y1c, x2c, y2c, ac = _cols(br3[i])
        x1r, y1r, x2r, y2r, ar = _rows(bc3[i])
        M = _sup_mask(x1c, y1c, x2c, y2c, ac,
                      x1r, y1r, x2r, y2r, ar) * upper
        valid = (lane + i * _B < _N).astype(jnp.bfloat16)
        pre = keep[i].astype(jnp.bfloat16) * valid
        kb = _fixpoint(pre, M)
        keep[i] = kb.astype(_F32)

        def sweep1(j, x1c=x1c, y1c=y1c, x2c=x2c, y2c=y2c, ac=ac, kb=kb):
            jx1, jy1, jx2, jy2, ja = _rows(bc3[j])
            g = _sup_mask(x1c, y1c, x2c, y2c, ac, jx1, jy1, jx2, jy2, ja)
            sup = jnp.dot(kb, g, preferred_element_type=_F32)
            keep[j] = keep[j] * jnp.where(sup < 0.5, 1.0, 0.0).astype(_F32)

        nj = _NB - 1 - i
        un = 4
        if nj > 0:
            def sweepu(t, carry, i=i, sweep1=sweep1):
                base = i + 1 + un * t
                for u in range(un):
                    sweep1(base + u)
                return carry

            if nj // un > 0:
                jax.lax.fori_loop(0, nj // un, sweepu, 0)
            for u in range(nj % un):
                sweep1(i + 1 + un * (nj // un) + u)

    def nkbody(j, t):
        valid = jnp.where(lane + j * _B < _N, 1.0, 0.0).astype(_F32)
        return t + jnp.sum(keep[j] * valid)

    nk = jax.lax.fori_loop(0, _NB, nkbody, jnp.float32(0.0))

    runk = jnp.float32(0.0)
    runs = jnp.float32(0.0)
    for j in range(_NB):
        valid = jnp.where(lane + j * _B < _N, 1.0, 0.0).astype(_F32)
        kp = keep[j] * valid
        sp = (1.0 - keep[j]) * valid
        prefk = jnp.dot(kp.astype(jnp.bfloat16), su,
                        preferred_element_type=_F32) + runk
        prefs = jnp.dot(sp.astype(jnp.bfloat16), su,
                        preferred_element_type=_F32) + runs
        rank2[0:1, j * _B:(j + 1) * _B] = jnp.where(
            kp > 0.5, prefk, jnp.where(sp > 0.5, nk + prefs, 3.0e7))
        conf2[0:1, j * _B:(j + 1) * _B] = 1.0 / (1.0 + jnp.exp(-s3[j]))
        runk = runk + jnp.sum(kp)
        runs = runs + jnp.sum(sp)


def _head_kernel(rank2, conf2, feats, w, bias, out):
    rrow = rank2[0:1, :]
    crow = conf2[0:1, :]
    fall = feats[:, :]
    for kc in range(_KP // _B):
        kcol = (jax.lax.broadcasted_iota(jnp.int32, (_B, 1), 0)
                .astype(_F32) + float(kc * _B))
        oh = ((rrow == kcol).astype(_F32) * crow).astype(jnp.bfloat16)
        acc = jnp.dot(oh, fall, preferred_element_type=_F32)
        res = jnp.dot(acc, w[:, :],
                      preferred_element_type=_F32) + bias[0:1, :]
        out[kc * _B:(kc + 1) * _B, :] = res


def kernel(boxes, scores, feats, cls_W, cls_b):
    order = jnp.argsort(-scores)
    f16 = feats.astype(jnp.bfloat16)
    fpk = jax.lax.bitcast_convert_type(f16.reshape(_N, _C // 2, 2), _F32)
    bsc = jnp.concatenate([boxes, scores[:, None]], axis=1)
    ps = bsc[order]
    fsg = fpk[order]
    fs = jax.lax.bitcast_convert_type(fsg, jnp.bfloat16).reshape(_N, _C)
    bs = ps[:, :4]
    ss = ps[:, 4]
    pad = _NP - _N
    bsp = jnp.concatenate([bs, jnp.zeros((pad, 4), _F32)], axis=0)
    ssp = jnp.concatenate([ss, jnp.zeros((pad,), _F32)], axis=0)
    fsp = jnp.concatenate([fs, jnp.zeros((pad, _C), jnp.bfloat16)], axis=0)
    br3 = bsp.reshape(_NB, _B, 4)
    bc3 = jnp.transpose(br3, (0, 2, 1))
    s3 = ssp.reshape(_NB, 1, _B)
    rank2, conf2 = pl.pallas_call(
        _nms_kernel,
        out_shape=[
            jax.ShapeDtypeStruct((1, _NP), _F32),
            jax.ShapeDtypeStruct((1, _NP), _F32),
        ],
        scratch_shapes=[pltpu.VMEM((_NB, 1, _B), _F32)],
        interpret=False,
    )(bc3, br3, s3)
    out = pl.pallas_call(
        _head_kernel,
        out_shape=jax.ShapeDtypeStruct((_KP, _NCLS), _F32),
        interpret=False,
    )(rank2, conf2, fsp, cls_W, cls_b.reshape(1, _NCLS))
    return out[:_K]
